# baseline (speedup 1.0000x reference)
import jax
import jax.numpy as jnp
import numpy as np
from jax import lax
from jax.experimental import pallas as pl
from jax.experimental.pallas import tpu as pltpu
from jax.experimental.pallas import tpu_sc as plsc

N = 10000
D = 128
NUM_GRAPHS = 64
NUM_ELEMENTS = 100
NUM_KEYS = NUM_GRAPHS * NUM_ELEMENTS
HALF_KEYS = NUM_KEYS // 2
NUM_TILES = 16
CHUNK = 128
MAX_TILE_CHUNKS = 5
ZB_ROWS = MAX_TILE_CHUNKS * CHUNK
ACC_ROWS = 3328
ZERO_ROWS = ACC_ROWS // NUM_TILES
OUT_ROWS_PER_TILE = HALF_KEYS // NUM_TILES


def _sc_body(out_hbm, z_hbm, b_hbm, split_hbm, zeros_hbm, o_hbm,
             rb0, rb1, zbuf, bbuf, keybuf, splitbuf, acc,
             sem0, sem1, semz, semzb):
    c = lax.axis_index("c")
    s = lax.axis_index("s")

    pltpu.sync_copy(split_hbm, splitbuf)
    sv = splitbuf[...]
    split = sv[0]
    for _i in range(1, 16):
        split = split + sv[_i]
    split8 = (split // 8) * 8
    is0 = c == 0
    cnt = jnp.where(is0, split, N - split8)
    base = jnp.where(is0, 0, split8)
    row_lo = jnp.where(is0, 0, split)
    row_hi = jnp.where(is0, split, N)
    m = (cnt + CHUNK - 1) // CHUNK
    cs = (m * s) // NUM_TILES
    ce = (m * (s + 1)) // NUM_TILES
    nch = ce - cs
    trash = HALF_KEYS + s
    key_base = c * HALF_KEYS
    zb_base = jnp.minimum(base + cs * CHUNK, N - ZB_ROWS)

    def _load_addr(k):
        return jnp.minimum(base + (cs + k) * CHUNK, N - CHUNK)

    def _start(k, rb, sem):
        pltpu.async_copy(out_hbm.at[pl.ds(_load_addr(k), CHUNK)], rb, sem)

    def _wait(rb, sem):
        pltpu.make_async_copy(out_hbm.at[pl.ds(0, CHUNK)], rb, sem).wait()

    def _process(k, rb):
        start_l = base + (cs + k) * CHUNK
        a = _load_addr(k)
        off = a - zb_base
        win_lo = jnp.maximum(start_l, row_lo)
        win_hi = jnp.minimum(start_l + CHUNK, row_hi)
        for j in range(CHUNK // 16):
            zv = zbuf[pl.ds(off + j * 16, 16)]
            bv = bbuf[pl.ds(off + j * 16, 16)]
            grow = a + j * 16 + lax.iota(jnp.int32, 16)
            key = bv * NUM_ELEMENTS + zv - 1 - key_base
            valid = ((grow >= win_lo) & (grow < win_hi)
                     & (key >= 0) & (key < HALF_KEYS))
            keybuf[pl.ds(j * 16, 16)] = jnp.where(valid, key, trash)
        pltpu.sync_copy(rb, acc.at[keybuf], add=True)

    _start(0, rb0, sem0)
    dz = pltpu.async_copy(z_hbm.at[pl.ds(zb_base, ZB_ROWS)], zbuf, semzb)
    db = pltpu.async_copy(b_hbm.at[pl.ds(zb_base, ZB_ROWS)], bbuf, semzb)
    zd = pltpu.async_copy(zeros_hbm, acc.at[pl.ds(s * ZERO_ROWS, ZERO_ROWS)],
                          semz)
    with jax.named_scope("zero_wait"):
        zd.wait()
    with jax.named_scope("zero_barrier"):
        plsc.subcore_barrier()
        dz.wait()
        db.wait()

    def _pair(i, carry):
        k0 = 2 * i
        _wait(rb0, sem0)
        _start(k0 + 1, rb1, sem1)

        @pl.when(k0 < nch)
        def _():
            _process(k0, rb0)

        _wait(rb1, sem1)
        _start(k0 + 2, rb0, sem0)

        @pl.when(k0 + 1 < nch)
        def _():
            _process(k0 + 1, rb1)

        return carry

    with jax.named_scope("scatter_loop"):
        lax.fori_loop(0, (nch + 1) // 2, _pair, 0)
        _wait(rb0, sem0)
    with jax.named_scope("final_barrier"):
        plsc.subcore_barrier()

    with jax.named_scope("readout"):
        obase = s * OUT_ROWS_PER_TILE
        pltpu.sync_copy(acc.at[pl.ds(obase, OUT_ROWS_PER_TILE)],
                        o_hbm.at[pl.ds(c * HALF_KEYS + obase, OUT_ROWS_PER_TILE)])


@jax.jit
def _pool_call(out, z32, b32):
    splits = jnp.sum((b32 < NUM_GRAPHS // 2).astype(jnp.int32).reshape(-1, 16),
                     axis=0, dtype=jnp.int32)
    zeros = np.zeros((ZERO_ROWS, D), np.float32)
    mesh = plsc.VectorSubcoreMesh(core_axis_name="c", subcore_axis_name="s")
    return pl.kernel(
        _sc_body,
        out_type=jax.ShapeDtypeStruct((NUM_KEYS, D), jnp.float32),
        mesh=mesh,
        scratch_types=[
            pltpu.VMEM((CHUNK, D), jnp.float32),
            pltpu.VMEM((CHUNK, D), jnp.float32),
            pltpu.VMEM((ZB_ROWS,), jnp.int32),
            pltpu.VMEM((ZB_ROWS,), jnp.int32),
            pltpu.VMEM((CHUNK,), jnp.int32),
            pltpu.VMEM((16,), jnp.int32),
            pltpu.VMEM_SHARED((ACC_ROWS, D), jnp.float32),
            pltpu.SemaphoreType.DMA,
            pltpu.SemaphoreType.DMA,
            pltpu.SemaphoreType.DMA,
            pltpu.SemaphoreType.DMA,
        ],
    )(out, z32, b32, splits, zeros)


def kernel(out, z, batch):
    pooled = _pool_call(out, z.astype(jnp.int32), batch.astype(jnp.int32))
    return pooled.reshape(NUM_GRAPHS, NUM_ELEMENTS * D)


# speedup vs baseline: 1.0625x; 1.0625x over previous
import jax
import jax.numpy as jnp
from jax import lax
from jax.experimental import pallas as pl
from jax.experimental.pallas import tpu as pltpu
from jax.experimental.pallas import tpu_sc as plsc

N = 10000
D = 128
NUM_GRAPHS = 64
NUM_ELEMENTS = 100
NUM_KEYS = NUM_GRAPHS * NUM_ELEMENTS
HALF_KEYS = NUM_KEYS // 2
NUM_TILES = 16
CHUNK = 128
MAX_TILE_CHUNKS = 5
ZB_ROWS = MAX_TILE_CHUNKS * CHUNK
ACC_ROWS = 3328
ZERO_ROWS = ACC_ROWS // NUM_TILES
ZBLOCK = 16
OUT_ROWS_PER_TILE = HALF_KEYS // NUM_TILES


def _sc_body(out_hbm, z_hbm, b_hbm, split_hbm, o_hbm,
             rb0, rb1, zbuf, bbuf, keybuf, splitbuf, zstage, acc,
             sem0, sem1, semzb):
    c = lax.axis_index("c")
    s = lax.axis_index("s")

    pltpu.sync_copy(split_hbm, splitbuf)
    sv = splitbuf[...]
    split = sv[0]
    for _i in range(1, 16):
        split = split + sv[_i]
    split8 = (split // 8) * 8
    is0 = c == 0
    cnt = jnp.where(is0, split, N - split8)
    base = jnp.where(is0, 0, split8)
    row_lo = jnp.where(is0, 0, split)
    row_hi = jnp.where(is0, split, N)
    m = (cnt + CHUNK - 1) // CHUNK
    cs = (m * s) // NUM_TILES
    ce = (m * (s + 1)) // NUM_TILES
    nch = ce - cs
    trash = HALF_KEYS + s
    key_base = c * HALF_KEYS
    zb_base = jnp.minimum(base + cs * CHUNK, N - ZB_ROWS)

    def _load_addr(k):
        return jnp.minimum(base + (cs + k) * CHUNK, N - CHUNK)

    def _start(k, rb, sem):
        pltpu.async_copy(out_hbm.at[pl.ds(_load_addr(k), CHUNK)], rb, sem)

    def _wait(rb, sem):
        pltpu.make_async_copy(out_hbm.at[pl.ds(0, CHUNK)], rb, sem).wait()

    def _process(k, rb):
        start_l = base + (cs + k) * CHUNK
        a = _load_addr(k)
        off = a - zb_base
        win_lo = jnp.maximum(start_l, row_lo)
        win_hi = jnp.minimum(start_l + CHUNK, row_hi)
        for j in range(CHUNK // 16):
            zv = zbuf[pl.ds(off + j * 16, 16)]
            bv = bbuf[pl.ds(off + j * 16, 16)]
            grow = a + j * 16 + lax.iota(jnp.int32, 16)
            key = bv * NUM_ELEMENTS + zv - 1 - key_base
            valid = ((grow >= win_lo) & (grow < win_hi)
                     & (key >= 0) & (key < HALF_KEYS))
            keybuf[pl.ds(j * 16, 16)] = jnp.where(valid, key, trash)
        pltpu.sync_copy(rb, acc.at[keybuf], add=True)

    _start(0, rb0, sem0)
    dz = pltpu.async_copy(z_hbm.at[pl.ds(zb_base, ZB_ROWS)], zbuf, semzb)
    db = pltpu.async_copy(b_hbm.at[pl.ds(zb_base, ZB_ROWS)], bbuf, semzb)
    for r in range(ZBLOCK):
        for j in range(D // 16):
            zstage[r, pl.ds(j * 16, 16)] = jnp.zeros((16,), jnp.float32)
    zbase = s * ZERO_ROWS
    for r in range(ZERO_ROWS // ZBLOCK):
        pltpu.sync_copy(zstage, acc.at[pl.ds(zbase + r * ZBLOCK, ZBLOCK)])
    plsc.subcore_barrier()
    dz.wait()
    db.wait()

    def _pair(i, carry):
        k0 = 2 * i
        _wait(rb0, sem0)
        _start(k0 + 1, rb1, sem1)

        @pl.when(k0 < nch)
        def _():
            _process(k0, rb0)

        _wait(rb1, sem1)
        _start(k0 + 2, rb0, sem0)

        @pl.when(k0 + 1 < nch)
        def _():
            _process(k0 + 1, rb1)

        return carry

    lax.fori_loop(0, (nch + 1) // 2, _pair, 0)
    _wait(rb0, sem0)
    plsc.subcore_barrier()

    obase = s * OUT_ROWS_PER_TILE
    pltpu.sync_copy(acc.at[pl.ds(obase, OUT_ROWS_PER_TILE)],
                    o_hbm.at[pl.ds(c * HALF_KEYS + obase, OUT_ROWS_PER_TILE)])


@jax.jit
def _pool_call(out, z32, b32):
    splits = jnp.sum((b32 < NUM_GRAPHS // 2).astype(jnp.int32).reshape(-1, 16),
                     axis=0, dtype=jnp.int32)
    mesh = plsc.VectorSubcoreMesh(core_axis_name="c", subcore_axis_name="s")
    return pl.kernel(
        _sc_body,
        out_type=jax.ShapeDtypeStruct((NUM_KEYS, D), jnp.float32),
        mesh=mesh,
        scratch_types=[
            pltpu.VMEM((CHUNK, D), jnp.float32),
            pltpu.VMEM((CHUNK, D), jnp.float32),
            pltpu.VMEM((ZB_ROWS,), jnp.int32),
            pltpu.VMEM((ZB_ROWS,), jnp.int32),
            pltpu.VMEM((CHUNK,), jnp.int32),
            pltpu.VMEM((16,), jnp.int32),
            pltpu.VMEM((ZBLOCK, D), jnp.float32),
            pltpu.VMEM_SHARED((ACC_ROWS, D), jnp.float32),
            pltpu.SemaphoreType.DMA,
            pltpu.SemaphoreType.DMA,
            pltpu.SemaphoreType.DMA,
        ],
    )(out, z32, b32, splits)


def kernel(out, z, batch):
    pooled = _pool_call(out, z.astype(jnp.int32), batch.astype(jnp.int32))
    return pooled.reshape(NUM_GRAPHS, NUM_ELEMENTS * D)
